# baseline (speedup 1.0000x reference)
import functools

import jax
import jax.numpy as jnp
from jax import lax
from jax.experimental import pallas as pl
from jax.experimental.pallas import tpu as pltpu
from jax.experimental.pallas import tpu_sc as plsc

MEM_DIM = 512
IN_DIM = 512
N_LEAF = 8192
N_CHILD = 4096
B = 16
S = 32
P = B * S
NWORKERS = 32
PER_W = P // NWORKERS
OFF_CW = N_LEAF
OFF_CI0 = N_LEAF + N_CHILD
OFF_CI1 = N_LEAF + 2 * N_CHILD


@functools.cache
def _sc_gather_make():
    mesh = plsc.VectorSubcoreMesh(core_axis_name="c", subcore_axis_name="s")
    f32 = jnp.float32
    i32 = jnp.int32
    out_type = (
        jax.ShapeDtypeStruct((3 * P, IN_DIM), f32),
        jax.ShapeDtypeStruct((4 * P,), f32),
    )
    scratch = [
        pltpu.VMEM((PER_W,), i32),
        pltpu.VMEM((4 * PER_W,), i32),
        pltpu.VMEM((4 * PER_W, 128), i32),
        pltpu.VMEM((2 * PER_W,), i32),
        pltpu.VMEM((2 * PER_W, 128), i32),
        pltpu.VMEM((3 * PER_W,), i32),
        pltpu.VMEM((3 * PER_W, IN_DIM), f32),
        pltpu.VMEM((4 * PER_W,), f32),
        pltpu.SemaphoreType.DMA,
    ]

    @functools.partial(pl.kernel, mesh=mesh, out_type=out_type,
                       scratch_types=scratch,
                       compiler_params=pltpu.CompilerParams(
                           needs_layout_passes=False))
    def sc_gather(ctx_hbm, tbl_hbm, embed_hbm,
                  rows_out, mk_out,
                  cidx_v, i1_v, g1_v, i2_v, g2_v, ri_v, rows_v, mk_v, sem):
        wid = lax.axis_index("s") * 2 + lax.axis_index("c")
        base = wid * PER_W
        pltpu.sync_copy(ctx_hbm.at[pl.ds(base, PER_W)], cidx_v)

        cidx = cidx_v[...]
        is_child = cidx > N_LEAF
        is_leaf = (cidx > 0) & (cidx <= N_LEAF)
        n_safe = jnp.where(is_child, cidx - (1 + N_LEAF), 0)
        leaf_i = jnp.where(is_leaf, cidx - 1, 0)
        iota = lax.iota(jnp.int32, PER_W)
        e0 = OFF_CW + n_safe
        e1 = leaf_i
        e2 = OFF_CI0 + n_safe
        e3 = OFF_CI1 + n_safe
        i1_v[pl.ds(0, PER_W)] = lax.shift_right_logical(e0, 7)
        i1_v[pl.ds(PER_W, PER_W)] = lax.shift_right_logical(e1, 7)
        i1_v[pl.ds(2 * PER_W, PER_W)] = lax.shift_right_logical(e2, 7)
        i1_v[pl.ds(3 * PER_W, PER_W)] = lax.shift_right_logical(e3, 7)
        pltpu.async_copy(tbl_hbm.at[i1_v], g1_v, sem).wait()

        cww = plsc.load_gather(g1_v, [iota, e0 & 127])
        lww = plsc.load_gather(g1_v, [iota + PER_W, e1 & 127])
        ci0 = plsc.load_gather(g1_v, [iota + 2 * PER_W, e2 & 127])
        ci1 = plsc.load_gather(g1_v, [iota + 3 * PER_W, e3 & 127])
        wx = jnp.where(is_child, cww, lww)
        m0 = is_child & (ci0 > 0)
        m1 = is_child & (ci1 > 0)
        ri_v[pl.ds(0, PER_W)] = wx
        e4 = jnp.where(m0, ci0 - 1, 0)
        e5 = jnp.where(m1, ci1 - 1, 0)
        i2_v[pl.ds(0, PER_W)] = lax.shift_right_logical(e4, 7)
        i2_v[pl.ds(PER_W, PER_W)] = lax.shift_right_logical(e5, 7)
        lvl2 = pltpu.async_copy(tbl_hbm.at[i2_v], g2_v, sem)

        one = jnp.float32(1.0)
        zero = jnp.float32(0.0)
        slot = lax.iota(jnp.int32, PER_W) * 4
        plsc.store_scatter(mk_v, [slot], jnp.where(is_child | is_leaf, one, zero))
        plsc.store_scatter(mk_v, [slot + 1], jnp.where(m0, one, zero))
        plsc.store_scatter(mk_v, [slot + 2], jnp.where(m1, one, zero))
        plsc.store_scatter(mk_v, [slot + 3], jnp.where(is_child, one, zero))
        cm = pltpu.async_copy(mk_v, mk_out.at[pl.ds(4 * base, 4 * PER_W)], sem)

        lvl2.wait()
        ri_v[pl.ds(PER_W, PER_W)] = plsc.load_gather(g2_v, [iota, e4 & 127])
        ri_v[pl.ds(2 * PER_W, PER_W)] = plsc.load_gather(
            g2_v, [iota + PER_W, e5 & 127])
        pltpu.async_copy(embed_hbm.at[ri_v], rows_v, sem).wait()
        co = pltpu.async_copy(rows_v, rows_out.at[pl.ds(3 * base, 3 * PER_W)], sem)
        cm.wait()
        co.wait()

    return sc_gather


def _tc_body(rows_ref, mk_ref,
             ioux_W_ref, iouh_W_ref, iou_b_ref,
             wih_f_ref, wih_b_ref, whh_f_ref, whh_b_ref,
             bih_f_ref, bih_b_ref, bhh_f_ref, bhh_b_ref,
             out_ref, gif_ref, gib_ref):
    H = MEM_DIM
    dn = (((1,), (1,)), ((), ()))

    x_raw = rows_ref[:, 0].reshape(P, IN_DIM)
    c0_raw = rows_ref[:, 1].reshape(P, IN_DIM)
    c1_raw = rows_ref[:, 2].reshape(P, IN_DIM)
    X = x_raw * mk_ref[:, 0:1]
    HS = c0_raw * mk_ref[:, 1:2] + c1_raw * mk_ref[:, 2:3]
    iou = (lax.dot_general(X, ioux_W_ref[...], dn)
           + lax.dot_general(HS, iouh_W_ref[...], dn)
           + iou_b_ref[...])
    i = jax.nn.sigmoid(iou[:, :H])
    o = jax.nn.sigmoid(iou[:, H:2 * H])
    u = jnp.tanh(iou[:, 2 * H:])
    h_op = o * jnp.tanh(i * u)
    mc = mk_ref[:, 3:4]
    enc = mc * h_op + (1.0 - mc) * X

    gif_ref[...] = (lax.dot_general(enc, wih_f_ref[...], dn)
                    + bih_f_ref[...]).reshape(S, B, 3 * H)
    gib_ref[...] = (lax.dot_general(enc, wih_b_ref[...], dn)
                    + bih_b_ref[...]).reshape(S, B, 3 * H)
    out_ref[...] = jnp.zeros((S, B, H), jnp.float32)

    whh_f = whh_f_ref[...]
    whh_b = whh_b_ref[...]
    bhh_f = bhh_f_ref[...]
    bhh_b = bhh_b_ref[...]

    def gru_step(gi, gh, h):
        r = jax.nn.sigmoid(gi[:, :H] + gh[:, :H])
        z = jax.nn.sigmoid(gi[:, H:2 * H] + gh[:, H:2 * H])
        n = jnp.tanh(gi[:, 2 * H:] + r * gh[:, 2 * H:])
        return (1.0 - z) * n + z * h

    def step(t, carry):
        h_f, h_b = carry
        gh_f = lax.dot_general(h_f, whh_f, dn) + bhh_f
        gh_b = lax.dot_general(h_b, whh_b, dn) + bhh_b
        h_f = gru_step(gif_ref[t], gh_f, h_f)
        h_b = gru_step(gib_ref[S - 1 - t], gh_b, h_b)
        out_ref[pl.ds(t, 1)] += h_f[None]
        out_ref[pl.ds(S - 1 - t, 1)] += h_b[None]
        return h_f, h_b

    h0 = jnp.zeros((B, MEM_DIM), jnp.float32)
    lax.fori_loop(0, S, step, (h0, h0))


def _tc_call(rows4, mk4,
             ioux_W, iouh_W, iou_b, wih_f, wih_b, whh_f, whh_b,
             bih_f, bih_b, bhh_f, bhh_b):
    return pl.pallas_call(
        _tc_body,
        out_shape=jax.ShapeDtypeStruct((S, B, MEM_DIM), jnp.float32),
        scratch_shapes=[
            pltpu.VMEM((S, B, 3 * MEM_DIM), jnp.float32),
            pltpu.VMEM((S, B, 3 * MEM_DIM), jnp.float32),
        ],
    )(rows4, mk4,
      ioux_W, iouh_W, iou_b, wih_f, wih_b, whh_f, whh_b,
      bih_f, bih_b, bhh_f, bhh_b)


def kernel(embed, leaf_word_idx, child_word_idx, child_idx, contexts_idx,
           ioux_W, ioux_b, iouh_W, iouh_b, fx_W, fx_b, fh_W, fh_b,
           Wih_f, Whh_f, bih_f, bhh_f, Wih_b, Whh_b, bih_b, bhh_b):
    ctx_sb = contexts_idx.T.reshape(-1).astype(jnp.int32)
    tbl = jnp.concatenate([
        leaf_word_idx.astype(jnp.int32),
        child_word_idx.astype(jnp.int32),
        child_idx.astype(jnp.int32).reshape(-1)]).reshape(-1, 128)
    rows_blk, mk_flat = _sc_gather_make()(ctx_sb, tbl, embed)

    out = _tc_call(
        rows_blk.reshape(NWORKERS, 3, PER_W, IN_DIM), mk_flat.reshape(P, 4),
        ioux_W, iouh_W, (ioux_b + iouh_b).reshape(1, 3 * MEM_DIM),
        Wih_f, Wih_b, Whh_f, Whh_b,
        bih_f.reshape(1, 3 * MEM_DIM), bih_b.reshape(1, 3 * MEM_DIM),
        bhh_f.reshape(1, 3 * MEM_DIM), bhh_b.reshape(1, 3 * MEM_DIM))
    return out.transpose(1, 0, 2)


# speedup vs baseline: 1.1998x; 1.1998x over previous
import functools

import jax
import jax.numpy as jnp
from jax import lax
from jax.experimental import pallas as pl
from jax.experimental.pallas import tpu as pltpu
from jax.experimental.pallas import tpu_sc as plsc

MEM_DIM = 512
IN_DIM = 512
N_LEAF = 8192
N_CHILD = 4096
B = 16
S = 32
P = B * S
NWORKERS = 32
PER_W = P // NWORKERS
OFF_CW = N_LEAF
OFF_CI0 = N_LEAF + N_CHILD
OFF_CI1 = N_LEAF + 2 * N_CHILD


@functools.cache
def _sc_gather_make():
    mesh = plsc.VectorSubcoreMesh(core_axis_name="c", subcore_axis_name="s")
    f32 = jnp.float32
    i32 = jnp.int32
    out_type = (
        jax.ShapeDtypeStruct((3 * P, IN_DIM), f32),
        jax.ShapeDtypeStruct((4 * P,), f32),
    )
    scratch = [
        pltpu.VMEM((PER_W,), i32),
        pltpu.VMEM((PER_W,), i32),
        pltpu.VMEM((PER_W,), i32),
        pltpu.VMEM((PER_W,), i32),
        pltpu.VMEM((PER_W,), i32),
        pltpu.VMEM((PER_W,), i32),
        pltpu.VMEM((PER_W,), i32),
        pltpu.VMEM((3 * PER_W,), i32),
        pltpu.VMEM((3 * PER_W, IN_DIM), f32),
        pltpu.VMEM((4 * PER_W,), f32),
        pltpu.SemaphoreType.DMA,
        pltpu.SemaphoreType.DMA,
    ]

    @functools.partial(pl.kernel, mesh=mesh, out_type=out_type,
                       scratch_types=scratch,
                       compiler_params=pltpu.CompilerParams(
                           needs_layout_passes=False))
    def sc_gather(ctx_hbm, tbl_hbm, embed_hbm,
                  rows_out, mk_out,
                  cidx_v, cwv_v, lwv_v, ci0_v, ci1_v, w0_v, w1_v,
                  ri_v, rows_v, mk_v, sem, osem):
        wid = lax.axis_index("s") * 2 + lax.axis_index("c")
        base = wid * PER_W
        pltpu.sync_copy(ctx_hbm.at[pl.ds(base, PER_W)], cidx_v)

        cidx = cidx_v[...]
        is_child = cidx > N_LEAF
        is_leaf = (cidx > 0) & (cidx <= N_LEAF)
        n_safe = jnp.where(is_child, cidx - (1 + N_LEAF), 0)
        leaf_i = jnp.where(is_leaf, cidx - 1, 0)
        g1 = pltpu.async_copy(tbl_hbm.at[OFF_CW + n_safe], cwv_v, sem)
        g2 = pltpu.async_copy(tbl_hbm.at[leaf_i], lwv_v, sem)
        g3 = pltpu.async_copy(tbl_hbm.at[OFF_CI0 + n_safe], ci0_v, sem)
        g4 = pltpu.async_copy(tbl_hbm.at[OFF_CI1 + n_safe], ci1_v, sem)
        g1.wait()
        g2.wait()
        g3.wait()
        g4.wait()

        wx = jnp.where(is_child, cwv_v[...], lwv_v[...])
        ci0 = ci0_v[...]
        ci1 = ci1_v[...]
        m0 = is_child & (ci0 > 0)
        m1 = is_child & (ci1 > 0)
        g5 = pltpu.async_copy(tbl_hbm.at[jnp.where(m0, ci0 - 1, 0)], w0_v, sem)
        g6 = pltpu.async_copy(tbl_hbm.at[jnp.where(m1, ci1 - 1, 0)], w1_v, sem)
        ri_v[pl.ds(0, PER_W)] = wx

        one = jnp.float32(1.0)
        zero = jnp.float32(0.0)
        slot = lax.iota(jnp.int32, PER_W) * 4
        plsc.store_scatter(mk_v, [slot], jnp.where(is_child | is_leaf, one, zero))
        plsc.store_scatter(mk_v, [slot + 1], jnp.where(m0, one, zero))
        plsc.store_scatter(mk_v, [slot + 2], jnp.where(m1, one, zero))
        plsc.store_scatter(mk_v, [slot + 3], jnp.where(is_child, one, zero))
        cm = pltpu.async_copy(mk_v, mk_out.at[pl.ds(4 * base, 4 * PER_W)], osem)

        g5.wait()
        g6.wait()
        ri_v[pl.ds(PER_W, PER_W)] = w0_v[...]
        ri_v[pl.ds(2 * PER_W, PER_W)] = w1_v[...]
        pltpu.async_copy(embed_hbm.at[ri_v], rows_v, sem).wait()
        co = pltpu.async_copy(rows_v, rows_out.at[pl.ds(3 * base, 3 * PER_W)],
                              osem)
        cm.wait()
        co.wait()

    return sc_gather


def _tc_body(rows_ref, mk_ref,
             ioux_W_ref, iouh_W_ref, iou_b_ref,
             wih_f_ref, wih_b_ref, whh_f_ref, whh_b_ref,
             bih_f_ref, bih_b_ref, bhh_f_ref, bhh_b_ref,
             out_ref, gif_ref, gib_ref):
    H = MEM_DIM
    dn = (((1,), (1,)), ((), ()))

    x_raw = rows_ref[:, 0].reshape(P, IN_DIM)
    c0_raw = rows_ref[:, 1].reshape(P, IN_DIM)
    c1_raw = rows_ref[:, 2].reshape(P, IN_DIM)
    X = x_raw * mk_ref[:, 0:1]
    HS = c0_raw * mk_ref[:, 1:2] + c1_raw * mk_ref[:, 2:3]
    iou = (lax.dot_general(X, ioux_W_ref[...], dn)
           + lax.dot_general(HS, iouh_W_ref[...], dn)
           + iou_b_ref[...])
    i = jax.nn.sigmoid(iou[:, :H])
    o = jax.nn.sigmoid(iou[:, H:2 * H])
    u = jnp.tanh(iou[:, 2 * H:])
    h_op = o * jnp.tanh(i * u)
    mc = mk_ref[:, 3:4]
    enc = mc * h_op + (1.0 - mc) * X

    gif_ref[...] = (lax.dot_general(enc, wih_f_ref[...], dn)
                    + bih_f_ref[...]).reshape(S, B, 3 * H)
    gib_ref[...] = (lax.dot_general(enc, wih_b_ref[...], dn)
                    + bih_b_ref[...]).reshape(S, B, 3 * H)
    out_ref[...] = jnp.zeros((S, B, H), jnp.float32)

    whh_f = whh_f_ref[...]
    whh_b = whh_b_ref[...]
    bhh_f = bhh_f_ref[...]
    bhh_b = bhh_b_ref[...]

    def gru_step(gi, gh, h):
        r = jax.nn.sigmoid(gi[:, :H] + gh[:, :H])
        z = jax.nn.sigmoid(gi[:, H:2 * H] + gh[:, H:2 * H])
        n = jnp.tanh(gi[:, 2 * H:] + r * gh[:, 2 * H:])
        return (1.0 - z) * n + z * h

    def step(t, carry):
        h_f, h_b = carry
        gh_f = lax.dot_general(h_f, whh_f, dn) + bhh_f
        gh_b = lax.dot_general(h_b, whh_b, dn) + bhh_b
        h_f = gru_step(gif_ref[t], gh_f, h_f)
        h_b = gru_step(gib_ref[S - 1 - t], gh_b, h_b)
        out_ref[pl.ds(t, 1)] += h_f[None]
        out_ref[pl.ds(S - 1 - t, 1)] += h_b[None]
        return h_f, h_b

    h0 = jnp.zeros((B, MEM_DIM), jnp.float32)
    lax.fori_loop(0, S, step, (h0, h0))


def _tc_call(rows4, mk4,
             ioux_W, iouh_W, iou_b, wih_f, wih_b, whh_f, whh_b,
             bih_f, bih_b, bhh_f, bhh_b):
    return pl.pallas_call(
        _tc_body,
        out_shape=jax.ShapeDtypeStruct((S, B, MEM_DIM), jnp.float32),
        scratch_shapes=[
            pltpu.VMEM((S, B, 3 * MEM_DIM), jnp.float32),
            pltpu.VMEM((S, B, 3 * MEM_DIM), jnp.float32),
        ],
    )(rows4, mk4,
      ioux_W, iouh_W, iou_b, wih_f, wih_b, whh_f, whh_b,
      bih_f, bih_b, bhh_f, bhh_b)


def kernel(embed, leaf_word_idx, child_word_idx, child_idx, contexts_idx,
           ioux_W, ioux_b, iouh_W, iouh_b, fx_W, fx_b, fh_W, fh_b,
           Wih_f, Whh_f, bih_f, bhh_f, Wih_b, Whh_b, bih_b, bhh_b):
    ctx_sb = contexts_idx.T.reshape(-1).astype(jnp.int32)
    tbl = jnp.concatenate([
        leaf_word_idx.astype(jnp.int32),
        child_word_idx.astype(jnp.int32),
        child_idx.astype(jnp.int32).reshape(-1)])
    rows_blk, mk_flat = _sc_gather_make()(ctx_sb, tbl, embed)

    out = _tc_call(
        rows_blk.reshape(NWORKERS, 3, PER_W, IN_DIM), mk_flat.reshape(P, 4),
        ioux_W, iouh_W, (ioux_b + iouh_b).reshape(1, 3 * MEM_DIM),
        Wih_f, Wih_b, Whh_f, Whh_b,
        bih_f.reshape(1, 3 * MEM_DIM), bih_b.reshape(1, 3 * MEM_DIM),
        bhh_f.reshape(1, 3 * MEM_DIM), bhh_b.reshape(1, 3 * MEM_DIM))
    return out.transpose(1, 0, 2)


# speedup vs baseline: 1.2023x; 1.0020x over previous
import functools

import jax
import jax.numpy as jnp
from jax import lax
from jax.experimental import pallas as pl
from jax.experimental.pallas import tpu as pltpu
from jax.experimental.pallas import tpu_sc as plsc

MEM_DIM = 512
IN_DIM = 512
N_LEAF = 8192
N_CHILD = 4096
B = 16
S = 32
P = B * S
NWORKERS = 32
PER_W = P // NWORKERS
OFF_CW = N_LEAF
OFF_CI0 = N_LEAF + N_CHILD
OFF_CI1 = N_LEAF + 2 * N_CHILD


@functools.cache
def _sc_gather_make():
    mesh = plsc.VectorSubcoreMesh(core_axis_name="c", subcore_axis_name="s")
    f32 = jnp.float32
    i32 = jnp.int32
    out_type = (
        jax.ShapeDtypeStruct((3 * P, IN_DIM), f32),
        jax.ShapeDtypeStruct((4 * P,), f32),
    )
    scratch = [
        pltpu.VMEM((PER_W,), i32),
        pltpu.VMEM((PER_W,), i32),
        pltpu.VMEM((PER_W,), i32),
        pltpu.VMEM((PER_W,), i32),
        pltpu.VMEM((PER_W,), i32),
        pltpu.VMEM((PER_W,), i32),
        pltpu.VMEM((PER_W,), i32),
        pltpu.VMEM((3 * PER_W,), i32),
        pltpu.VMEM((3 * PER_W, IN_DIM), f32),
        pltpu.VMEM((4 * PER_W,), f32),
        pltpu.SemaphoreType.DMA,
        pltpu.SemaphoreType.DMA,
    ]

    @functools.partial(pl.kernel, mesh=mesh, out_type=out_type,
                       scratch_types=scratch,
                       compiler_params=pltpu.CompilerParams(
                           needs_layout_passes=False))
    def sc_gather(ctx_hbm, tbl_hbm, embed_hbm,
                  rows_out, mk_out,
                  cidx_v, cwv_v, lwv_v, ci0_v, ci1_v, w0_v, w1_v,
                  ri_v, rows_v, mk_v, sem, osem):
        wid = lax.axis_index("s") * 2 + lax.axis_index("c")
        base = wid * PER_W
        pltpu.sync_copy(ctx_hbm.at[pl.ds(base, PER_W)], cidx_v)

        cidx = cidx_v[...]
        is_child = cidx > N_LEAF
        is_leaf = (cidx > 0) & (cidx <= N_LEAF)
        n_safe = jnp.where(is_child, cidx - (1 + N_LEAF), 0)
        leaf_i = jnp.where(is_leaf, cidx - 1, 0)
        g1 = pltpu.async_copy(tbl_hbm.at[OFF_CW + n_safe], cwv_v, sem)
        g2 = pltpu.async_copy(tbl_hbm.at[leaf_i], lwv_v, sem)
        g3 = pltpu.async_copy(tbl_hbm.at[OFF_CI0 + n_safe], ci0_v, sem)
        g4 = pltpu.async_copy(tbl_hbm.at[OFF_CI1 + n_safe], ci1_v, sem)
        g1.wait()
        g2.wait()
        g3.wait()
        g4.wait()

        wx = jnp.where(is_child, cwv_v[...], lwv_v[...])
        ci0 = ci0_v[...]
        ci1 = ci1_v[...]
        m0 = is_child & (ci0 > 0)
        m1 = is_child & (ci1 > 0)
        g5 = pltpu.async_copy(tbl_hbm.at[jnp.where(m0, ci0 - 1, 0)], w0_v, sem)
        g6 = pltpu.async_copy(tbl_hbm.at[jnp.where(m1, ci1 - 1, 0)], w1_v, sem)
        ri_v[pl.ds(0, PER_W)] = wx

        one = jnp.float32(1.0)
        zero = jnp.float32(0.0)
        slot = lax.iota(jnp.int32, PER_W) * 4
        plsc.store_scatter(mk_v, [slot], jnp.where(is_child | is_leaf, one, zero))
        plsc.store_scatter(mk_v, [slot + 1], jnp.where(m0, one, zero))
        plsc.store_scatter(mk_v, [slot + 2], jnp.where(m1, one, zero))
        plsc.store_scatter(mk_v, [slot + 3], jnp.where(is_child, one, zero))
        cm = pltpu.async_copy(mk_v, mk_out.at[pl.ds(4 * base, 4 * PER_W)], osem)

        g5.wait()
        g6.wait()
        ri_v[pl.ds(PER_W, PER_W)] = w0_v[...]
        ri_v[pl.ds(2 * PER_W, PER_W)] = w1_v[...]
        pltpu.async_copy(embed_hbm.at[ri_v], rows_v, sem).wait()
        co = pltpu.async_copy(rows_v, rows_out.at[pl.ds(3 * base, 3 * PER_W)],
                              osem)
        cm.wait()
        co.wait()

    return sc_gather


def _tc_body(rows_ref, mk_ref,
             ioux_W_ref, iouh_W_ref, iou_b_ref,
             wih_f_ref, wih_b_ref, whh_f_ref, whh_b_ref,
             bih_f_ref, bih_b_ref, bhh_f_ref, bhh_b_ref,
             out_ref, gif_ref, gib_ref, bwd_ref):
    H = MEM_DIM
    dn = (((1,), (1,)), ((), ()))

    x_raw = rows_ref[:, 0].reshape(P, IN_DIM)
    c0_raw = rows_ref[:, 1].reshape(P, IN_DIM)
    c1_raw = rows_ref[:, 2].reshape(P, IN_DIM)
    X = x_raw * mk_ref[:, 0:1]
    HS = c0_raw * mk_ref[:, 1:2] + c1_raw * mk_ref[:, 2:3]
    iou = (lax.dot_general(X, ioux_W_ref[...], dn)
           + lax.dot_general(HS, iouh_W_ref[...], dn)
           + iou_b_ref[...])
    i = jax.nn.sigmoid(iou[:, :H])
    o = jax.nn.sigmoid(iou[:, H:2 * H])
    u = jnp.tanh(iou[:, 2 * H:])
    h_op = o * jnp.tanh(i * u)
    mc = mk_ref[:, 3:4]
    enc = mc * h_op + (1.0 - mc) * X

    gif_ref[...] = (lax.dot_general(enc, wih_f_ref[...], dn)
                    + bih_f_ref[...]).reshape(S, B, 3 * H)
    gib_ref[...] = (lax.dot_general(enc, wih_b_ref[...], dn)
                    + bih_b_ref[...]).reshape(S, B, 3 * H)

    def step(t, carry):
        h_f, h_b = carry
        gi_f = gif_ref[t]
        gi_b = gib_ref[S - 1 - t]
        r_f = jax.nn.sigmoid(
            gi_f[:, :H] + lax.dot_general(h_f, whh_f_ref[:H], dn)
            + bhh_f_ref[:, :H])
        r_b = jax.nn.sigmoid(
            gi_b[:, :H] + lax.dot_general(h_b, whh_b_ref[:H], dn)
            + bhh_b_ref[:, :H])
        z_f = jax.nn.sigmoid(
            gi_f[:, H:2 * H] + lax.dot_general(h_f, whh_f_ref[H:2 * H], dn)
            + bhh_f_ref[:, H:2 * H])
        z_b = jax.nn.sigmoid(
            gi_b[:, H:2 * H] + lax.dot_general(h_b, whh_b_ref[H:2 * H], dn)
            + bhh_b_ref[:, H:2 * H])
        n_f = jnp.tanh(
            gi_f[:, 2 * H:]
            + r_f * (lax.dot_general(h_f, whh_f_ref[2 * H:], dn)
                     + bhh_f_ref[:, 2 * H:]))
        n_b = jnp.tanh(
            gi_b[:, 2 * H:]
            + r_b * (lax.dot_general(h_b, whh_b_ref[2 * H:], dn)
                     + bhh_b_ref[:, 2 * H:]))
        h_f = (1.0 - z_f) * n_f + z_f * h_f
        h_b = (1.0 - z_b) * n_b + z_b * h_b
        out_ref[pl.ds(t, 1)] = h_f[None]
        bwd_ref[pl.ds(S - 1 - t, 1)] = h_b[None]
        return h_f, h_b

    h0 = jnp.zeros((B, MEM_DIM), jnp.float32)
    lax.fori_loop(0, S, step, (h0, h0))
    out_ref[...] += bwd_ref[...]


def _tc_call(rows4, mk4,
             ioux_W, iouh_W, iou_b, wih_f, wih_b, whh_f, whh_b,
             bih_f, bih_b, bhh_f, bhh_b):
    return pl.pallas_call(
        _tc_body,
        out_shape=jax.ShapeDtypeStruct((S, B, MEM_DIM), jnp.float32),
        scratch_shapes=[
            pltpu.VMEM((S, B, 3 * MEM_DIM), jnp.float32),
            pltpu.VMEM((S, B, 3 * MEM_DIM), jnp.float32),
            pltpu.VMEM((S, B, MEM_DIM), jnp.float32),
        ],
    )(rows4, mk4,
      ioux_W, iouh_W, iou_b, wih_f, wih_b, whh_f, whh_b,
      bih_f, bih_b, bhh_f, bhh_b)


def kernel(embed, leaf_word_idx, child_word_idx, child_idx, contexts_idx,
           ioux_W, ioux_b, iouh_W, iouh_b, fx_W, fx_b, fh_W, fh_b,
           Wih_f, Whh_f, bih_f, bhh_f, Wih_b, Whh_b, bih_b, bhh_b):
    ctx_sb = contexts_idx.T.reshape(-1).astype(jnp.int32)
    tbl = jnp.concatenate([
        leaf_word_idx.astype(jnp.int32),
        child_word_idx.astype(jnp.int32),
        child_idx.astype(jnp.int32).reshape(-1)])
    rows_blk, mk_flat = _sc_gather_make()(ctx_sb, tbl, embed)

    out = _tc_call(
        rows_blk.reshape(NWORKERS, 3, PER_W, IN_DIM), mk_flat.reshape(P, 4),
        ioux_W, iouh_W, (ioux_b + iouh_b).reshape(1, 3 * MEM_DIM),
        Wih_f, Wih_b, Whh_f, Whh_b,
        bih_f.reshape(1, 3 * MEM_DIM), bih_b.reshape(1, 3 * MEM_DIM),
        bhh_f.reshape(1, 3 * MEM_DIM), bhh_b.reshape(1, 3 * MEM_DIM))
    return out.transpose(1, 0, 2)


# speedup vs baseline: 1.2372x; 1.0290x over previous
import functools

import jax
import jax.numpy as jnp
from jax import lax
from jax.experimental import pallas as pl
from jax.experimental.pallas import tpu as pltpu
from jax.experimental.pallas import tpu_sc as plsc

MEM_DIM = 512
IN_DIM = 512
N_LEAF = 8192
N_CHILD = 4096
B = 16
S = 32
P = B * S
NWORKERS = 32
PER_W = P // NWORKERS
OFF_CW = N_LEAF
OFF_CI0 = N_LEAF + N_CHILD
OFF_CI1 = N_LEAF + 2 * N_CHILD


@functools.cache
def _sc_gather_make():
    mesh = plsc.VectorSubcoreMesh(core_axis_name="c", subcore_axis_name="s")
    f32 = jnp.float32
    i32 = jnp.int32
    out_type = (
        jax.ShapeDtypeStruct((3 * P, IN_DIM), f32),
        jax.ShapeDtypeStruct((4 * P,), f32),
    )
    scratch = [
        pltpu.VMEM((PER_W,), i32),
        pltpu.VMEM((PER_W,), i32),
        pltpu.VMEM((PER_W,), i32),
        pltpu.VMEM((PER_W,), i32),
        pltpu.VMEM((PER_W,), i32),
        pltpu.VMEM((PER_W,), i32),
        pltpu.VMEM((PER_W,), i32),
        pltpu.VMEM((3 * PER_W,), i32),
        pltpu.VMEM((3 * PER_W, IN_DIM), f32),
        pltpu.VMEM((4 * PER_W,), f32),
        pltpu.SemaphoreType.DMA,
        pltpu.SemaphoreType.DMA,
    ]

    @functools.partial(pl.kernel, mesh=mesh, out_type=out_type,
                       scratch_types=scratch,
                       compiler_params=pltpu.CompilerParams(
                           needs_layout_passes=False))
    def sc_gather(ctx_hbm, tbl_hbm, embed_hbm,
                  rows_out, mk_out,
                  cidx_v, cwv_v, lwv_v, ci0_v, ci1_v, w0_v, w1_v,
                  ri_v, rows_v, mk_v, sem, osem):
        wid = lax.axis_index("s") * 2 + lax.axis_index("c")
        base = wid * PER_W
        pltpu.sync_copy(ctx_hbm.at[pl.ds(base, PER_W)], cidx_v)

        cidx = cidx_v[...]
        is_child = cidx > N_LEAF
        is_leaf = (cidx > 0) & (cidx <= N_LEAF)
        n_safe = jnp.where(is_child, cidx - (1 + N_LEAF), 0)
        leaf_i = jnp.where(is_leaf, cidx - 1, 0)
        g1 = pltpu.async_copy(tbl_hbm.at[OFF_CW + n_safe], cwv_v, sem)
        g2 = pltpu.async_copy(tbl_hbm.at[leaf_i], lwv_v, sem)
        g3 = pltpu.async_copy(tbl_hbm.at[OFF_CI0 + n_safe], ci0_v, sem)
        g4 = pltpu.async_copy(tbl_hbm.at[OFF_CI1 + n_safe], ci1_v, sem)
        g1.wait()
        g2.wait()
        g3.wait()
        g4.wait()

        wx = jnp.where(is_child, cwv_v[...], lwv_v[...])
        ci0 = ci0_v[...]
        ci1 = ci1_v[...]
        m0 = is_child & (ci0 > 0)
        m1 = is_child & (ci1 > 0)
        g5 = pltpu.async_copy(tbl_hbm.at[jnp.where(m0, ci0 - 1, 0)], w0_v, sem)
        g6 = pltpu.async_copy(tbl_hbm.at[jnp.where(m1, ci1 - 1, 0)], w1_v, sem)
        ri_v[pl.ds(0, PER_W)] = wx

        one = jnp.float32(1.0)
        zero = jnp.float32(0.0)
        slot = lax.iota(jnp.int32, PER_W) * 4
        plsc.store_scatter(mk_v, [slot], jnp.where(is_child | is_leaf, one, zero))
        plsc.store_scatter(mk_v, [slot + 1], jnp.where(m0, one, zero))
        plsc.store_scatter(mk_v, [slot + 2], jnp.where(m1, one, zero))
        plsc.store_scatter(mk_v, [slot + 3], jnp.where(is_child, one, zero))
        cm = pltpu.async_copy(mk_v, mk_out.at[pl.ds(4 * base, 4 * PER_W)], osem)

        g5.wait()
        g6.wait()
        ri_v[pl.ds(PER_W, PER_W)] = w0_v[...]
        ri_v[pl.ds(2 * PER_W, PER_W)] = w1_v[...]
        pltpu.async_copy(embed_hbm.at[ri_v], rows_v, sem).wait()
        co = pltpu.async_copy(rows_v, rows_out.at[pl.ds(3 * base, 3 * PER_W)],
                              osem)
        cm.wait()
        co.wait()

    return sc_gather


def _tc_body(rows_ref, mk_ref,
             ioux_W_ref, iouh_W_ref, iou_b_ref,
             wih_f_ref, wih_b_ref, whh_f_ref, whh_b_ref,
             bih_f_ref, bih_b_ref, bhh_f_ref, bhh_b_ref,
             out_ref, gif_ref, gib_ref, bwd_ref):
    H = MEM_DIM
    dn = (((1,), (1,)), ((), ()))

    x_raw = rows_ref[:, 0].reshape(P, IN_DIM)
    c0_raw = rows_ref[:, 1].reshape(P, IN_DIM)
    c1_raw = rows_ref[:, 2].reshape(P, IN_DIM)
    X = x_raw * mk_ref[:, 0:1]
    HS = c0_raw * mk_ref[:, 1:2] + c1_raw * mk_ref[:, 2:3]
    iou = (lax.dot_general(X, ioux_W_ref[...], dn)
           + lax.dot_general(HS, iouh_W_ref[...], dn)
           + iou_b_ref[...])
    i = jax.nn.sigmoid(iou[:, :H])
    o = jax.nn.sigmoid(iou[:, H:2 * H])
    u = jnp.tanh(iou[:, 2 * H:])
    h_op = o * jnp.tanh(i * u)
    mc = mk_ref[:, 3:4]
    enc = mc * h_op + (1.0 - mc) * X

    gif_ref[...] = (lax.dot_general(enc, wih_f_ref[...], dn)
                    + bih_f_ref[...]).reshape(S, B, 3 * H)
    gib_ref[...] = (lax.dot_general(enc, wih_b_ref[...], dn)
                    + bih_b_ref[...]).reshape(S, B, 3 * H)

    def step(t, carry):
        h_f, h_b = carry
        gi_f = gif_ref[t]
        gi_b = gib_ref[S - 1 - t]
        r_f = jax.nn.sigmoid(
            gi_f[:, :H] + lax.dot_general(h_f, whh_f_ref[:H], dn)
            + bhh_f_ref[:, :H])
        r_b = jax.nn.sigmoid(
            gi_b[:, :H] + lax.dot_general(h_b, whh_b_ref[:H], dn)
            + bhh_b_ref[:, :H])
        z_f = jax.nn.sigmoid(
            gi_f[:, H:2 * H] + lax.dot_general(h_f, whh_f_ref[H:2 * H], dn)
            + bhh_f_ref[:, H:2 * H])
        z_b = jax.nn.sigmoid(
            gi_b[:, H:2 * H] + lax.dot_general(h_b, whh_b_ref[H:2 * H], dn)
            + bhh_b_ref[:, H:2 * H])
        n_f = jnp.tanh(
            gi_f[:, 2 * H:]
            + r_f * (lax.dot_general(h_f, whh_f_ref[2 * H:], dn)
                     + bhh_f_ref[:, 2 * H:]))
        n_b = jnp.tanh(
            gi_b[:, 2 * H:]
            + r_b * (lax.dot_general(h_b, whh_b_ref[2 * H:], dn)
                     + bhh_b_ref[:, 2 * H:]))
        h_f = (1.0 - z_f) * n_f + z_f * h_f
        h_b = (1.0 - z_b) * n_b + z_b * h_b
        out_ref[:, pl.ds(t, 1)] = h_f[:, None]
        bwd_ref[:, pl.ds(S - 1 - t, 1)] = h_b[:, None]
        return h_f, h_b

    h0 = jnp.zeros((B, MEM_DIM), jnp.float32)
    lax.fori_loop(0, S, step, (h0, h0))
    out_ref[...] += bwd_ref[...]


def _tc_call(rows4, mk4,
             ioux_W, iouh_W, iou_b, wih_f, wih_b, whh_f, whh_b,
             bih_f, bih_b, bhh_f, bhh_b):
    return pl.pallas_call(
        _tc_body,
        out_shape=jax.ShapeDtypeStruct((B, S, MEM_DIM), jnp.float32),
        scratch_shapes=[
            pltpu.VMEM((S, B, 3 * MEM_DIM), jnp.float32),
            pltpu.VMEM((S, B, 3 * MEM_DIM), jnp.float32),
            pltpu.VMEM((B, S, MEM_DIM), jnp.float32),
        ],
    )(rows4, mk4,
      ioux_W, iouh_W, iou_b, wih_f, wih_b, whh_f, whh_b,
      bih_f, bih_b, bhh_f, bhh_b)


def kernel(embed, leaf_word_idx, child_word_idx, child_idx, contexts_idx,
           ioux_W, ioux_b, iouh_W, iouh_b, fx_W, fx_b, fh_W, fh_b,
           Wih_f, Whh_f, bih_f, bhh_f, Wih_b, Whh_b, bih_b, bhh_b):
    ctx_sb = contexts_idx.T.reshape(-1).astype(jnp.int32)
    tbl = jnp.concatenate([
        leaf_word_idx.astype(jnp.int32),
        child_word_idx.astype(jnp.int32),
        child_idx.astype(jnp.int32).reshape(-1)])
    rows_blk, mk_flat = _sc_gather_make()(ctx_sb, tbl, embed)

    out = _tc_call(
        rows_blk.reshape(NWORKERS, 3, PER_W, IN_DIM), mk_flat.reshape(P, 4),
        ioux_W, iouh_W, (ioux_b + iouh_b).reshape(1, 3 * MEM_DIM),
        Wih_f, Wih_b, Whh_f, Whh_b,
        bih_f.reshape(1, 3 * MEM_DIM), bih_b.reshape(1, 3 * MEM_DIM),
        bhh_f.reshape(1, 3 * MEM_DIM), bhh_b.reshape(1, 3 * MEM_DIM))
    return out


# speedup vs baseline: 1.2452x; 1.0065x over previous
import functools

import jax
import jax.numpy as jnp
from jax import lax
from jax.experimental import pallas as pl
from jax.experimental.pallas import tpu as pltpu
from jax.experimental.pallas import tpu_sc as plsc

MEM_DIM = 512
IN_DIM = 512
N_LEAF = 8192
N_CHILD = 4096
B = 16
S = 32
P = B * S
NWORKERS = 32
PER_W = P // NWORKERS

@functools.cache
def _sc_gather_make():
    mesh = plsc.VectorSubcoreMesh(core_axis_name="c", subcore_axis_name="s")
    f32 = jnp.float32
    i32 = jnp.int32
    out_type = (
        jax.ShapeDtypeStruct((3 * P, IN_DIM), f32),
        jax.ShapeDtypeStruct((4 * P,), f32),
    )
    scratch = [
        pltpu.VMEM((PER_W,), i32),
        pltpu.VMEM((PER_W,), i32),
        pltpu.VMEM((PER_W,), i32),
        pltpu.VMEM((PER_W,), i32),
        pltpu.VMEM((PER_W,), i32),
        pltpu.VMEM((PER_W,), i32),
        pltpu.VMEM((PER_W,), i32),
        pltpu.VMEM((3 * PER_W,), i32),
        pltpu.VMEM((3 * PER_W, IN_DIM), f32),
        pltpu.VMEM((4 * PER_W,), f32),
        pltpu.SemaphoreType.DMA,
        pltpu.SemaphoreType.DMA,
    ]

    @functools.partial(pl.kernel, mesh=mesh, out_type=out_type,
                       scratch_types=scratch,
                       compiler_params=pltpu.CompilerParams(
                           needs_layout_passes=False))
    def sc_gather(ctx_hbm, lw_hbm, cw_hbm, ci_hbm, embed_hbm,
                  rows_out, mk_out,
                  cidx_v, cwv_v, lwv_v, ci0_v, ci1_v, w0_v, w1_v,
                  ri_v, rows_v, mk_v, sem, osem):
        wid = lax.axis_index("s") * 2 + lax.axis_index("c")
        base = wid * PER_W
        iota = lax.iota(jnp.int32, PER_W)
        pltpu.async_copy(ctx_hbm.at[iota * S + wid], cidx_v, sem).wait()

        cidx = cidx_v[...]
        is_child = cidx > N_LEAF
        is_leaf = (cidx > 0) & (cidx <= N_LEAF)
        n_safe = jnp.where(is_child, cidx - (1 + N_LEAF), 0)
        leaf_i = jnp.where(is_leaf, cidx - 1, 0)
        g1 = pltpu.async_copy(cw_hbm.at[n_safe], cwv_v, sem)
        g2 = pltpu.async_copy(lw_hbm.at[leaf_i], lwv_v, sem)
        g3 = pltpu.async_copy(ci_hbm.at[n_safe], ci0_v, sem)
        g4 = pltpu.async_copy(ci_hbm.at[N_CHILD + n_safe], ci1_v, sem)
        g1.wait()
        g2.wait()
        g3.wait()
        g4.wait()

        wx = jnp.where(is_child, cwv_v[...], lwv_v[...])
        ci0 = ci0_v[...]
        ci1 = ci1_v[...]
        m0 = is_child & (ci0 > 0)
        m1 = is_child & (ci1 > 0)
        g5 = pltpu.async_copy(lw_hbm.at[jnp.where(m0, ci0 - 1, 0)], w0_v, sem)
        g6 = pltpu.async_copy(lw_hbm.at[jnp.where(m1, ci1 - 1, 0)], w1_v, sem)
        ri_v[pl.ds(0, PER_W)] = wx

        one = jnp.float32(1.0)
        zero = jnp.float32(0.0)
        slot = iota * 4
        plsc.store_scatter(mk_v, [slot], jnp.where(is_child | is_leaf, one, zero))
        plsc.store_scatter(mk_v, [slot + 1], jnp.where(m0, one, zero))
        plsc.store_scatter(mk_v, [slot + 2], jnp.where(m1, one, zero))
        plsc.store_scatter(mk_v, [slot + 3], jnp.where(is_child, one, zero))
        cm = pltpu.async_copy(mk_v, mk_out.at[pl.ds(4 * base, 4 * PER_W)], osem)

        g5.wait()
        g6.wait()
        ri_v[pl.ds(PER_W, PER_W)] = w0_v[...]
        ri_v[pl.ds(2 * PER_W, PER_W)] = w1_v[...]
        pltpu.async_copy(embed_hbm.at[ri_v], rows_v, sem).wait()
        co = pltpu.async_copy(rows_v, rows_out.at[pl.ds(3 * base, 3 * PER_W)],
                              osem)
        cm.wait()
        co.wait()

    return sc_gather


def _tc_body(rows_ref, mk_ref,
             ioux_W_ref, iouh_W_ref, ioux_b_ref, iouh_b_ref,
             wih_f_ref, wih_b_ref, whh_f_ref, whh_b_ref,
             bih_f_ref, bih_b_ref, bhh_f_ref, bhh_b_ref,
             out_ref, gif_ref, gib_ref, bwd_ref):
    H = MEM_DIM
    dn = (((1,), (1,)), ((), ()))

    x_raw = rows_ref[:, 0].reshape(P, IN_DIM)
    c0_raw = rows_ref[:, 1].reshape(P, IN_DIM)
    c1_raw = rows_ref[:, 2].reshape(P, IN_DIM)
    X = x_raw * mk_ref[:, 0:1]
    HS = c0_raw * mk_ref[:, 1:2] + c1_raw * mk_ref[:, 2:3]
    iou = (lax.dot_general(X, ioux_W_ref[...], dn)
           + lax.dot_general(HS, iouh_W_ref[...], dn)
           + (ioux_b_ref[...] + iouh_b_ref[...]))
    i = jax.nn.sigmoid(iou[:, :H])
    o = jax.nn.sigmoid(iou[:, H:2 * H])
    u = jnp.tanh(iou[:, 2 * H:])
    h_op = o * jnp.tanh(i * u)
    mc = mk_ref[:, 3:4]
    enc = mc * h_op + (1.0 - mc) * X

    gif_ref[...] = (lax.dot_general(enc, wih_f_ref[...], dn)
                    + bih_f_ref[...]).reshape(S, B, 3 * H)
    gib_ref[...] = (lax.dot_general(enc, wih_b_ref[...], dn)
                    + bih_b_ref[...]).reshape(S, B, 3 * H)

    def step(t, carry):
        h_f, h_b = carry
        gi_f = gif_ref[t]
        gi_b = gib_ref[S - 1 - t]
        r_f = jax.nn.sigmoid(
            gi_f[:, :H] + lax.dot_general(h_f, whh_f_ref[:H], dn)
            + bhh_f_ref[:, :H])
        r_b = jax.nn.sigmoid(
            gi_b[:, :H] + lax.dot_general(h_b, whh_b_ref[:H], dn)
            + bhh_b_ref[:, :H])
        z_f = jax.nn.sigmoid(
            gi_f[:, H:2 * H] + lax.dot_general(h_f, whh_f_ref[H:2 * H], dn)
            + bhh_f_ref[:, H:2 * H])
        z_b = jax.nn.sigmoid(
            gi_b[:, H:2 * H] + lax.dot_general(h_b, whh_b_ref[H:2 * H], dn)
            + bhh_b_ref[:, H:2 * H])
        n_f = jnp.tanh(
            gi_f[:, 2 * H:]
            + r_f * (lax.dot_general(h_f, whh_f_ref[2 * H:], dn)
                     + bhh_f_ref[:, 2 * H:]))
        n_b = jnp.tanh(
            gi_b[:, 2 * H:]
            + r_b * (lax.dot_general(h_b, whh_b_ref[2 * H:], dn)
                     + bhh_b_ref[:, 2 * H:]))
        h_f = (1.0 - z_f) * n_f + z_f * h_f
        h_b = (1.0 - z_b) * n_b + z_b * h_b
        out_ref[:, pl.ds(t, 1)] = h_f[:, None]
        bwd_ref[:, pl.ds(S - 1 - t, 1)] = h_b[:, None]
        return h_f, h_b

    h0 = jnp.zeros((B, MEM_DIM), jnp.float32)
    lax.fori_loop(0, S, step, (h0, h0))
    out_ref[...] += bwd_ref[...]


def _tc_call(rows4, mk4,
             ioux_W, iouh_W, ioux_b, iouh_b, wih_f, wih_b, whh_f, whh_b,
             bih_f, bih_b, bhh_f, bhh_b):
    return pl.pallas_call(
        _tc_body,
        out_shape=jax.ShapeDtypeStruct((B, S, MEM_DIM), jnp.float32),
        scratch_shapes=[
            pltpu.VMEM((S, B, 3 * MEM_DIM), jnp.float32),
            pltpu.VMEM((S, B, 3 * MEM_DIM), jnp.float32),
            pltpu.VMEM((B, S, MEM_DIM), jnp.float32),
        ],
    )(rows4, mk4,
      ioux_W, iouh_W, ioux_b, iouh_b, wih_f, wih_b, whh_f, whh_b,
      bih_f, bih_b, bhh_f, bhh_b)


def kernel(embed, leaf_word_idx, child_word_idx, child_idx, contexts_idx,
           ioux_W, ioux_b, iouh_W, iouh_b, fx_W, fx_b, fh_W, fh_b,
           Wih_f, Whh_f, bih_f, bhh_f, Wih_b, Whh_b, bih_b, bhh_b):
    rows_blk, mk_flat = _sc_gather_make()(
        contexts_idx.reshape(-1).astype(jnp.int32),
        leaf_word_idx.astype(jnp.int32),
        child_word_idx.astype(jnp.int32),
        child_idx.astype(jnp.int32).reshape(-1), embed)

    out = _tc_call(
        rows_blk.reshape(NWORKERS, 3, PER_W, IN_DIM), mk_flat.reshape(P, 4),
        ioux_W, iouh_W,
        ioux_b.reshape(1, 3 * MEM_DIM), iouh_b.reshape(1, 3 * MEM_DIM),
        Wih_f, Wih_b, Whh_f, Whh_b,
        bih_f.reshape(1, 3 * MEM_DIM), bih_b.reshape(1, 3 * MEM_DIM),
        bhh_f.reshape(1, 3 * MEM_DIM), bhh_b.reshape(1, 3 * MEM_DIM))
    return out


# speedup vs baseline: 1.2594x; 1.0114x over previous
import functools

import jax
import jax.numpy as jnp
from jax import lax
from jax.experimental import pallas as pl
from jax.experimental.pallas import tpu as pltpu
from jax.experimental.pallas import tpu_sc as plsc

MEM_DIM = 512
IN_DIM = 512
N_LEAF = 8192
N_CHILD = 4096
B = 16
S = 32
P = B * S
NWORKERS = 32
PER_W = P // NWORKERS

@functools.cache
def _sc_gather_make():
    mesh = plsc.VectorSubcoreMesh(core_axis_name="c", subcore_axis_name="s")
    f32 = jnp.float32
    i32 = jnp.int32
    out_type = (
        jax.ShapeDtypeStruct((3 * P, IN_DIM), f32),
        jax.ShapeDtypeStruct((4 * P,), f32),
    )
    scratch = [
        pltpu.VMEM((PER_W,), i32),
        pltpu.VMEM((PER_W,), i32),
        pltpu.VMEM((PER_W,), i32),
        pltpu.VMEM((PER_W,), i32),
        pltpu.VMEM((PER_W,), i32),
        pltpu.VMEM((PER_W,), i32),
        pltpu.VMEM((PER_W,), i32),
        pltpu.VMEM((3 * PER_W,), i32),
        pltpu.VMEM((3 * PER_W, IN_DIM), f32),
        pltpu.VMEM((4 * PER_W,), f32),
        pltpu.SemaphoreType.DMA,
        pltpu.SemaphoreType.DMA,
    ]

    @functools.partial(pl.kernel, mesh=mesh, out_type=out_type,
                       scratch_types=scratch,
                       compiler_params=pltpu.CompilerParams(
                           needs_layout_passes=False))
    def sc_gather(ctx_hbm, lw_hbm, cw_hbm, ci_hbm, embed_hbm,
                  rows_out, mk_out,
                  cidx_v, cwv_v, lwv_v, ci0_v, ci1_v, w0_v, w1_v,
                  ri_v, rows_v, mk_v, sem, osem):
        wid = lax.axis_index("s") * 2 + lax.axis_index("c")
        base = wid * PER_W
        iota = lax.iota(jnp.int32, PER_W)
        pltpu.async_copy(ctx_hbm.at[iota * S + wid], cidx_v, sem).wait()

        cidx = cidx_v[...]
        is_child = cidx > N_LEAF
        is_leaf = (cidx > 0) & (cidx <= N_LEAF)
        n_safe = jnp.where(is_child, cidx - (1 + N_LEAF), 0)
        leaf_i = jnp.where(is_leaf, cidx - 1, 0)
        g1 = pltpu.async_copy(cw_hbm.at[n_safe], cwv_v, sem)
        g2 = pltpu.async_copy(lw_hbm.at[leaf_i], lwv_v, sem)
        g3 = pltpu.async_copy(ci_hbm.at[n_safe], ci0_v, sem)
        g4 = pltpu.async_copy(ci_hbm.at[N_CHILD + n_safe], ci1_v, sem)
        g1.wait()
        g2.wait()
        g3.wait()
        g4.wait()

        wx = jnp.where(is_child, cwv_v[...], lwv_v[...])
        ci0 = ci0_v[...]
        ci1 = ci1_v[...]
        m0 = is_child & (ci0 > 0)
        m1 = is_child & (ci1 > 0)
        g5 = pltpu.async_copy(lw_hbm.at[jnp.where(m0, ci0 - 1, 0)], w0_v, sem)
        g6 = pltpu.async_copy(lw_hbm.at[jnp.where(m1, ci1 - 1, 0)], w1_v, sem)
        ri_v[pl.ds(0, PER_W)] = wx

        one = jnp.float32(1.0)
        zero = jnp.float32(0.0)
        slot = iota * 4
        plsc.store_scatter(mk_v, [slot], jnp.where(is_child | is_leaf, one, zero))
        plsc.store_scatter(mk_v, [slot + 1], jnp.where(m0, one, zero))
        plsc.store_scatter(mk_v, [slot + 2], jnp.where(m1, one, zero))
        plsc.store_scatter(mk_v, [slot + 3], jnp.where(is_child, one, zero))
        cm = pltpu.async_copy(mk_v, mk_out.at[pl.ds(4 * base, 4 * PER_W)], osem)

        g5.wait()
        g6.wait()
        ri_v[pl.ds(PER_W, PER_W)] = w0_v[...]
        ri_v[pl.ds(2 * PER_W, PER_W)] = w1_v[...]
        pltpu.async_copy(embed_hbm.at[ri_v], rows_v, sem).wait()
        co = pltpu.async_copy(rows_v, rows_out.at[pl.ds(3 * base, 3 * PER_W)],
                              osem)
        cm.wait()
        co.wait()

    return sc_gather


def _tc_body(rows_ref, mk_ref,
             ioux_W_ref, iouh_W_ref, ioux_b_ref, iouh_b_ref,
             wih_f_ref, wih_b_ref, whh_f_ref, whh_b_ref,
             bih_f_ref, bih_b_ref, bhh_f_ref, bhh_b_ref,
             out_ref, gif_ref, gib_ref, bwd_ref,
             wfv_ref, wbv_ref, hfv_ref, hbv_ref, wsem, hsem):
    H = MEM_DIM
    dn = (((1,), (1,)), ((), ()))

    cw1 = pltpu.async_copy(wih_f_ref, wfv_ref, wsem)
    cw2 = pltpu.async_copy(wih_b_ref, wbv_ref, wsem)
    ch1 = pltpu.async_copy(whh_f_ref, hfv_ref, hsem)
    ch2 = pltpu.async_copy(whh_b_ref, hbv_ref, hsem)

    x_raw = rows_ref[:, 0].reshape(P, IN_DIM)
    c0_raw = rows_ref[:, 1].reshape(P, IN_DIM)
    c1_raw = rows_ref[:, 2].reshape(P, IN_DIM)
    X = x_raw * mk_ref[:, 0:1]
    HS = c0_raw * mk_ref[:, 1:2] + c1_raw * mk_ref[:, 2:3]
    iou = (lax.dot_general(X, ioux_W_ref[...], dn)
           + lax.dot_general(HS, iouh_W_ref[...], dn)
           + (ioux_b_ref[...] + iouh_b_ref[...]))
    i = jax.nn.sigmoid(iou[:, :H])
    o = jax.nn.sigmoid(iou[:, H:2 * H])
    u = jnp.tanh(iou[:, 2 * H:])
    h_op = o * jnp.tanh(i * u)
    mc = mk_ref[:, 3:4]
    enc = mc * h_op + (1.0 - mc) * X

    cw1.wait()
    cw2.wait()
    gif_ref[...] = (lax.dot_general(enc, wfv_ref[...], dn)
                    + bih_f_ref[...]).reshape(S, B, 3 * H)
    gib_ref[...] = (lax.dot_general(enc, wbv_ref[...], dn)
                    + bih_b_ref[...]).reshape(S, B, 3 * H)
    ch1.wait()
    ch2.wait()

    def step(t, carry):
        h_f, h_b = carry
        gi_f = gif_ref[t]
        gi_b = gib_ref[S - 1 - t]
        r_f = jax.nn.sigmoid(
            gi_f[:, :H] + lax.dot_general(h_f, hfv_ref[:H], dn)
            + bhh_f_ref[:, :H])
        r_b = jax.nn.sigmoid(
            gi_b[:, :H] + lax.dot_general(h_b, hbv_ref[:H], dn)
            + bhh_b_ref[:, :H])
        z_f = jax.nn.sigmoid(
            gi_f[:, H:2 * H] + lax.dot_general(h_f, hfv_ref[H:2 * H], dn)
            + bhh_f_ref[:, H:2 * H])
        z_b = jax.nn.sigmoid(
            gi_b[:, H:2 * H] + lax.dot_general(h_b, hbv_ref[H:2 * H], dn)
            + bhh_b_ref[:, H:2 * H])
        n_f = jnp.tanh(
            gi_f[:, 2 * H:]
            + r_f * (lax.dot_general(h_f, hfv_ref[2 * H:], dn)
                     + bhh_f_ref[:, 2 * H:]))
        n_b = jnp.tanh(
            gi_b[:, 2 * H:]
            + r_b * (lax.dot_general(h_b, hbv_ref[2 * H:], dn)
                     + bhh_b_ref[:, 2 * H:]))
        h_f = (1.0 - z_f) * n_f + z_f * h_f
        h_b = (1.0 - z_b) * n_b + z_b * h_b
        out_ref[:, pl.ds(t, 1)] = h_f[:, None]
        bwd_ref[:, pl.ds(S - 1 - t, 1)] = h_b[:, None]
        return h_f, h_b

    h0 = jnp.zeros((B, MEM_DIM), jnp.float32)
    lax.fori_loop(0, S, step, (h0, h0))
    out_ref[...] += bwd_ref[...]


def _tc_call(rows4, mk4,
             ioux_W, iouh_W, ioux_b, iouh_b, wih_f, wih_b, whh_f, whh_b,
             bih_f, bih_b, bhh_f, bhh_b):
    vspec = pl.BlockSpec(memory_space=pltpu.MemorySpace.VMEM)
    hspec = pl.BlockSpec(memory_space=pltpu.MemorySpace.HBM)
    return pl.pallas_call(
        _tc_body,
        out_shape=jax.ShapeDtypeStruct((B, S, MEM_DIM), jnp.float32),
        in_specs=[vspec, vspec, vspec, vspec, vspec, vspec,
                  hspec, hspec, hspec, hspec,
                  vspec, vspec, vspec, vspec],
        scratch_shapes=[
            pltpu.VMEM((S, B, 3 * MEM_DIM), jnp.float32),
            pltpu.VMEM((S, B, 3 * MEM_DIM), jnp.float32),
            pltpu.VMEM((B, S, MEM_DIM), jnp.float32),
            pltpu.VMEM((3 * MEM_DIM, MEM_DIM), jnp.float32),
            pltpu.VMEM((3 * MEM_DIM, MEM_DIM), jnp.float32),
            pltpu.VMEM((3 * MEM_DIM, MEM_DIM), jnp.float32),
            pltpu.VMEM((3 * MEM_DIM, MEM_DIM), jnp.float32),
            pltpu.SemaphoreType.DMA,
            pltpu.SemaphoreType.DMA,
        ],
    )(rows4, mk4,
      ioux_W, iouh_W, ioux_b, iouh_b, wih_f, wih_b, whh_f, whh_b,
      bih_f, bih_b, bhh_f, bhh_b)


def kernel(embed, leaf_word_idx, child_word_idx, child_idx, contexts_idx,
           ioux_W, ioux_b, iouh_W, iouh_b, fx_W, fx_b, fh_W, fh_b,
           Wih_f, Whh_f, bih_f, bhh_f, Wih_b, Whh_b, bih_b, bhh_b):
    rows_blk, mk_flat = _sc_gather_make()(
        contexts_idx.reshape(-1).astype(jnp.int32),
        leaf_word_idx.astype(jnp.int32),
        child_word_idx.astype(jnp.int32),
        child_idx.astype(jnp.int32).reshape(-1), embed)

    out = _tc_call(
        rows_blk.reshape(NWORKERS, 3, PER_W, IN_DIM), mk_flat.reshape(P, 4),
        ioux_W, iouh_W,
        ioux_b.reshape(1, 3 * MEM_DIM), iouh_b.reshape(1, 3 * MEM_DIM),
        Wih_f, Wih_b, Whh_f, Whh_b,
        bih_f.reshape(1, 3 * MEM_DIM), bih_b.reshape(1, 3 * MEM_DIM),
        bhh_f.reshape(1, 3 * MEM_DIM), bhh_b.reshape(1, 3 * MEM_DIM))
    return out


# speedup vs baseline: 1.2879x; 1.0226x over previous
import functools

import jax
import jax.numpy as jnp
from jax import lax
from jax.experimental import pallas as pl
from jax.experimental.pallas import tpu as pltpu
from jax.experimental.pallas import tpu_sc as plsc

MEM_DIM = 512
IN_DIM = 512
N_LEAF = 8192
N_CHILD = 4096
B = 16
S = 32
P = B * S
NWORKERS = 32
PER_W = P // NWORKERS

@functools.cache
def _sc_gather_make():
    mesh = plsc.VectorSubcoreMesh(core_axis_name="c", subcore_axis_name="s")
    f32 = jnp.float32
    i32 = jnp.int32
    out_type = (
        jax.ShapeDtypeStruct((3 * P, IN_DIM), f32),
        jax.ShapeDtypeStruct((4 * P,), f32),
    )
    scratch = [
        pltpu.VMEM((PER_W,), i32),
        pltpu.VMEM((PER_W,), i32),
        pltpu.VMEM((PER_W,), i32),
        pltpu.VMEM((PER_W,), i32),
        pltpu.VMEM((PER_W,), i32),
        pltpu.VMEM((PER_W,), i32),
        pltpu.VMEM((PER_W,), i32),
        pltpu.VMEM((3 * PER_W,), i32),
        pltpu.VMEM((3 * PER_W, IN_DIM), f32),
        pltpu.VMEM((4 * PER_W,), f32),
        pltpu.SemaphoreType.DMA,
        pltpu.SemaphoreType.DMA,
    ]

    @functools.partial(pl.kernel, mesh=mesh, out_type=out_type,
                       scratch_types=scratch,
                       compiler_params=pltpu.CompilerParams(
                           needs_layout_passes=False))
    def sc_gather(ctx_hbm, lw_hbm, cw_hbm, ci_hbm, embed_hbm,
                  rows_out, mk_out,
                  cidx_v, cwv_v, lwv_v, ci0_v, ci1_v, w0_v, w1_v,
                  ri_v, rows_v, mk_v, sem, osem):
        wid = lax.axis_index("s") * 2 + lax.axis_index("c")
        base = wid * PER_W
        iota = lax.iota(jnp.int32, PER_W)
        pltpu.async_copy(ctx_hbm.at[iota * S + wid], cidx_v, sem).wait()

        cidx = cidx_v[...]
        is_child = cidx > N_LEAF
        is_leaf = (cidx > 0) & (cidx <= N_LEAF)
        n_safe = jnp.where(is_child, cidx - (1 + N_LEAF), 0)
        leaf_i = jnp.where(is_leaf, cidx - 1, 0)
        g1 = pltpu.async_copy(cw_hbm.at[n_safe], cwv_v, sem)
        g2 = pltpu.async_copy(lw_hbm.at[leaf_i], lwv_v, sem)
        g3 = pltpu.async_copy(ci_hbm.at[n_safe], ci0_v, sem)
        g4 = pltpu.async_copy(ci_hbm.at[N_CHILD + n_safe], ci1_v, sem)
        g1.wait()
        g2.wait()
        g3.wait()
        g4.wait()

        wx = jnp.where(is_child, cwv_v[...], lwv_v[...])
        ci0 = ci0_v[...]
        ci1 = ci1_v[...]
        m0 = is_child & (ci0 > 0)
        m1 = is_child & (ci1 > 0)
        g5 = pltpu.async_copy(lw_hbm.at[jnp.where(m0, ci0 - 1, 0)], w0_v, sem)
        g6 = pltpu.async_copy(lw_hbm.at[jnp.where(m1, ci1 - 1, 0)], w1_v, sem)
        ri_v[pl.ds(0, PER_W)] = wx

        one = jnp.float32(1.0)
        zero = jnp.float32(0.0)
        slot = iota * 4
        plsc.store_scatter(mk_v, [slot], jnp.where(is_child | is_leaf, one, zero))
        plsc.store_scatter(mk_v, [slot + 1], jnp.where(m0, one, zero))
        plsc.store_scatter(mk_v, [slot + 2], jnp.where(m1, one, zero))
        plsc.store_scatter(mk_v, [slot + 3], jnp.where(is_child, one, zero))
        cm = pltpu.async_copy(mk_v, mk_out.at[pl.ds(4 * base, 4 * PER_W)], osem)

        g5.wait()
        g6.wait()
        ri_v[pl.ds(PER_W, PER_W)] = w0_v[...]
        ri_v[pl.ds(2 * PER_W, PER_W)] = w1_v[...]
        pltpu.async_copy(embed_hbm.at[ri_v], rows_v, sem).wait()
        co = pltpu.async_copy(rows_v, rows_out.at[pl.ds(3 * base, 3 * PER_W)],
                              osem)
        cm.wait()
        co.wait()

    return sc_gather


def _tc_body(rows_ref, mk_ref,
             ioux_W_ref, iouh_W_ref, ioux_b_ref, iouh_b_ref,
             wih_f_ref, wih_b_ref, whh_f_ref, whh_b_ref,
             bih_f_ref, bih_b_ref, bhh_f_ref, bhh_b_ref,
             out_ref, gif_ref, gib_ref, bwd_ref,
             wfv_ref, wbv_ref, hfv_ref, hbv_ref, wsem, hsem):
    H = MEM_DIM
    dn = (((1,), (1,)), ((), ()))

    cw1 = pltpu.async_copy(wih_f_ref, wfv_ref, wsem)
    cw2 = pltpu.async_copy(wih_b_ref, wbv_ref, wsem)
    ch1 = pltpu.async_copy(whh_f_ref, hfv_ref, hsem)
    ch2 = pltpu.async_copy(whh_b_ref, hbv_ref, hsem)

    x_raw = rows_ref[:, 0].reshape(P, IN_DIM)
    c0_raw = rows_ref[:, 1].reshape(P, IN_DIM)
    c1_raw = rows_ref[:, 2].reshape(P, IN_DIM)
    X = x_raw * mk_ref[:, 0:1]
    HS = c0_raw * mk_ref[:, 1:2] + c1_raw * mk_ref[:, 2:3]
    iou = (lax.dot_general(X, ioux_W_ref[...], dn)
           + lax.dot_general(HS, iouh_W_ref[...], dn)
           + (ioux_b_ref[...] + iouh_b_ref[...]))
    i = jax.nn.sigmoid(iou[:, :H])
    o = jax.nn.sigmoid(iou[:, H:2 * H])
    u = jnp.tanh(iou[:, 2 * H:])
    h_op = o * jnp.tanh(i * u)
    mc = mk_ref[:, 3:4]
    enc = mc * h_op + (1.0 - mc) * X

    cw1.wait()
    cw2.wait()
    gif_ref[...] = (lax.dot_general(enc, wfv_ref[...], dn)
                    + bih_f_ref[...]).reshape(S, B, 3 * H)
    gib_ref[...] = (lax.dot_general(enc, wbv_ref[...], dn)
                    + bih_b_ref[...]).reshape(S, B, 3 * H)
    ch1.wait()
    ch2.wait()

    def step(t, carry):
        h_f, h_b = carry
        gi_f = gif_ref[t]
        gi_b = gib_ref[S - 1 - t]
        r_f = jax.nn.sigmoid(
            gi_f[:, :H] + lax.dot_general(h_f, hfv_ref[:H], dn)
            + bhh_f_ref[:, :H])
        r_b = jax.nn.sigmoid(
            gi_b[:, :H] + lax.dot_general(h_b, hbv_ref[:H], dn)
            + bhh_b_ref[:, :H])
        z_f = jax.nn.sigmoid(
            gi_f[:, H:2 * H] + lax.dot_general(h_f, hfv_ref[H:2 * H], dn)
            + bhh_f_ref[:, H:2 * H])
        z_b = jax.nn.sigmoid(
            gi_b[:, H:2 * H] + lax.dot_general(h_b, hbv_ref[H:2 * H], dn)
            + bhh_b_ref[:, H:2 * H])
        n_f = jnp.tanh(
            gi_f[:, 2 * H:]
            + r_f * (lax.dot_general(h_f, hfv_ref[2 * H:], dn)
                     + bhh_f_ref[:, 2 * H:]))
        n_b = jnp.tanh(
            gi_b[:, 2 * H:]
            + r_b * (lax.dot_general(h_b, hbv_ref[2 * H:], dn)
                     + bhh_b_ref[:, 2 * H:]))
        h_f = (1.0 - z_f) * n_f + z_f * h_f
        h_b = (1.0 - z_b) * n_b + z_b * h_b
        out_ref[:, pl.ds(t, 1)] = h_f[:, None]
        bwd_ref[:, pl.ds(S - 1 - t, 1)] = h_b[:, None]
        return h_f, h_b

    def step2(u, carry):
        return step(2 * u + 1, step(2 * u, carry))

    h0 = jnp.zeros((B, MEM_DIM), jnp.float32)
    lax.fori_loop(0, S // 2, step2, (h0, h0))
    out_ref[...] += bwd_ref[...]


def _tc_call(rows4, mk4,
             ioux_W, iouh_W, ioux_b, iouh_b, wih_f, wih_b, whh_f, whh_b,
             bih_f, bih_b, bhh_f, bhh_b):
    vspec = pl.BlockSpec(memory_space=pltpu.MemorySpace.VMEM)
    hspec = pl.BlockSpec(memory_space=pltpu.MemorySpace.HBM)
    return pl.pallas_call(
        _tc_body,
        out_shape=jax.ShapeDtypeStruct((B, S, MEM_DIM), jnp.float32),
        in_specs=[vspec, vspec, vspec, vspec, vspec, vspec,
                  hspec, hspec, hspec, hspec,
                  vspec, vspec, vspec, vspec],
        scratch_shapes=[
            pltpu.VMEM((S, B, 3 * MEM_DIM), jnp.float32),
            pltpu.VMEM((S, B, 3 * MEM_DIM), jnp.float32),
            pltpu.VMEM((B, S, MEM_DIM), jnp.float32),
            pltpu.VMEM((3 * MEM_DIM, MEM_DIM), jnp.float32),
            pltpu.VMEM((3 * MEM_DIM, MEM_DIM), jnp.float32),
            pltpu.VMEM((3 * MEM_DIM, MEM_DIM), jnp.float32),
            pltpu.VMEM((3 * MEM_DIM, MEM_DIM), jnp.float32),
            pltpu.SemaphoreType.DMA,
            pltpu.SemaphoreType.DMA,
        ],
    )(rows4, mk4,
      ioux_W, iouh_W, ioux_b, iouh_b, wih_f, wih_b, whh_f, whh_b,
      bih_f, bih_b, bhh_f, bhh_b)


def kernel(embed, leaf_word_idx, child_word_idx, child_idx, contexts_idx,
           ioux_W, ioux_b, iouh_W, iouh_b, fx_W, fx_b, fh_W, fh_b,
           Wih_f, Whh_f, bih_f, bhh_f, Wih_b, Whh_b, bih_b, bhh_b):
    rows_blk, mk_flat = _sc_gather_make()(
        contexts_idx.reshape(-1).astype(jnp.int32),
        leaf_word_idx.astype(jnp.int32),
        child_word_idx.astype(jnp.int32),
        child_idx.astype(jnp.int32).reshape(-1), embed)

    out = _tc_call(
        rows_blk.reshape(NWORKERS, 3, PER_W, IN_DIM), mk_flat.reshape(P, 4),
        ioux_W, iouh_W,
        ioux_b.reshape(1, 3 * MEM_DIM), iouh_b.reshape(1, 3 * MEM_DIM),
        Wih_f, Wih_b, Whh_f, Whh_b,
        bih_f.reshape(1, 3 * MEM_DIM), bih_b.reshape(1, 3 * MEM_DIM),
        bhh_f.reshape(1, 3 * MEM_DIM), bhh_b.reshape(1, 3 * MEM_DIM))
    return out


# speedup vs baseline: 1.3054x; 1.0136x over previous
import functools

import jax
import jax.numpy as jnp
from jax import lax
from jax.experimental import pallas as pl
from jax.experimental.pallas import tpu as pltpu
from jax.experimental.pallas import tpu_sc as plsc

MEM_DIM = 512
IN_DIM = 512
N_LEAF = 8192
N_CHILD = 4096
B = 16
S = 32
P = B * S
NWORKERS = 32
PER_W = P // NWORKERS

@functools.cache
def _sc_gather_make():
    mesh = plsc.VectorSubcoreMesh(core_axis_name="c", subcore_axis_name="s")
    f32 = jnp.float32
    i32 = jnp.int32
    out_type = (
        jax.ShapeDtypeStruct((3 * P, IN_DIM), f32),
        jax.ShapeDtypeStruct((4 * P,), f32),
    )
    scratch = [
        pltpu.VMEM((PER_W,), i32),
        pltpu.VMEM((PER_W,), i32),
        pltpu.VMEM((PER_W,), i32),
        pltpu.VMEM((PER_W,), i32),
        pltpu.VMEM((PER_W,), i32),
        pltpu.VMEM((PER_W,), i32),
        pltpu.VMEM((PER_W,), i32),
        pltpu.VMEM((3 * PER_W,), i32),
        pltpu.VMEM((3 * PER_W, IN_DIM), f32),
        pltpu.VMEM((4 * PER_W,), f32),
        pltpu.SemaphoreType.DMA,
        pltpu.SemaphoreType.DMA,
    ]

    @functools.partial(pl.kernel, mesh=mesh, out_type=out_type,
                       scratch_types=scratch,
                       compiler_params=pltpu.CompilerParams(
                           needs_layout_passes=False))
    def sc_gather(ctx_hbm, lw_hbm, cw_hbm, ci_hbm, embed_hbm,
                  rows_out, mk_out,
                  cidx_v, cwv_v, lwv_v, ci0_v, ci1_v, w0_v, w1_v,
                  ri_v, rows_v, mk_v, sem, osem):
        wid = lax.axis_index("s") * 2 + lax.axis_index("c")
        base = wid * PER_W
        iota = lax.iota(jnp.int32, PER_W)
        pltpu.async_copy(ctx_hbm.at[iota * S + wid], cidx_v, sem).wait()

        cidx = cidx_v[...]
        is_child = cidx > N_LEAF
        is_leaf = (cidx > 0) & (cidx <= N_LEAF)
        n_safe = jnp.where(is_child, cidx - (1 + N_LEAF), 0)
        leaf_i = jnp.where(is_leaf, cidx - 1, 0)
        g1 = pltpu.async_copy(cw_hbm.at[n_safe], cwv_v, sem)
        g2 = pltpu.async_copy(lw_hbm.at[leaf_i], lwv_v, sem)
        g3 = pltpu.async_copy(ci_hbm.at[n_safe], ci0_v, sem)
        g4 = pltpu.async_copy(ci_hbm.at[N_CHILD + n_safe], ci1_v, sem)
        g1.wait()
        g2.wait()
        g3.wait()
        g4.wait()

        wx = jnp.where(is_child, cwv_v[...], lwv_v[...])
        ci0 = ci0_v[...]
        ci1 = ci1_v[...]
        m0 = is_child & (ci0 > 0)
        m1 = is_child & (ci1 > 0)
        g5 = pltpu.async_copy(lw_hbm.at[jnp.where(m0, ci0 - 1, 0)], w0_v, sem)
        g6 = pltpu.async_copy(lw_hbm.at[jnp.where(m1, ci1 - 1, 0)], w1_v, sem)
        ri_v[pl.ds(0, PER_W)] = wx

        one = jnp.float32(1.0)
        zero = jnp.float32(0.0)
        slot = iota * 4
        plsc.store_scatter(mk_v, [slot], jnp.where(is_child | is_leaf, one, zero))
        plsc.store_scatter(mk_v, [slot + 1], jnp.where(m0, one, zero))
        plsc.store_scatter(mk_v, [slot + 2], jnp.where(m1, one, zero))
        plsc.store_scatter(mk_v, [slot + 3], jnp.where(is_child, one, zero))
        cm = pltpu.async_copy(mk_v, mk_out.at[pl.ds(4 * base, 4 * PER_W)], osem)

        g5.wait()
        g6.wait()
        ri_v[pl.ds(PER_W, PER_W)] = w0_v[...]
        ri_v[pl.ds(2 * PER_W, PER_W)] = w1_v[...]
        pltpu.async_copy(embed_hbm.at[ri_v], rows_v, sem).wait()
        co = pltpu.async_copy(rows_v, rows_out.at[pl.ds(3 * base, 3 * PER_W)],
                              osem)
        cm.wait()
        co.wait()

    return sc_gather


def _tc_body(rows_ref, mk_ref,
             ioux_W_ref, iouh_W_ref, ioux_b_ref, iouh_b_ref,
             wih_f_ref, wih_b_ref, whh_f_ref, whh_b_ref,
             bih_f_ref, bih_b_ref, bhh_f_ref, bhh_b_ref,
             out_ref, gif_ref, gib_ref, bwd_ref,
             wfv_ref, wbv_ref, hfv_ref, hbv_ref, wsem, hsem):
    H = MEM_DIM
    dn = (((1,), (1,)), ((), ()))

    cw1 = pltpu.async_copy(wih_f_ref, wfv_ref, wsem)
    cw2 = pltpu.async_copy(wih_b_ref, wbv_ref, wsem)
    ch1 = pltpu.async_copy(whh_f_ref, hfv_ref, hsem)
    ch2 = pltpu.async_copy(whh_b_ref, hbv_ref, hsem)

    x_raw = rows_ref[:, 0].reshape(P, IN_DIM)
    c0_raw = rows_ref[:, 1].reshape(P, IN_DIM)
    c1_raw = rows_ref[:, 2].reshape(P, IN_DIM)
    X = x_raw * mk_ref[:, 0:1]
    HS = c0_raw * mk_ref[:, 1:2] + c1_raw * mk_ref[:, 2:3]
    iou = (lax.dot_general(X, ioux_W_ref[...], dn)
           + lax.dot_general(HS, iouh_W_ref[...], dn)
           + (ioux_b_ref[...] + iouh_b_ref[...]))
    i = jax.nn.sigmoid(iou[:, :H])
    o = jax.nn.sigmoid(iou[:, H:2 * H])
    u = jnp.tanh(iou[:, 2 * H:])
    h_op = o * jnp.tanh(i * u)
    mc = mk_ref[:, 3:4]
    enc = mc * h_op + (1.0 - mc) * X

    cw1.wait()
    cw2.wait()
    gif_ref[...] = (lax.dot_general(enc, wfv_ref[...], dn)
                    + bih_f_ref[...]).reshape(S, B, 3 * H)
    gib_ref[...] = (lax.dot_general(enc, wbv_ref[...], dn)
                    + bih_b_ref[...]).reshape(S, B, 3 * H)
    ch1.wait()
    ch2.wait()

    def step(t, carry):
        h_f, h_b = carry
        gi_f = gif_ref[t]
        gi_b = gib_ref[S - 1 - t]
        r_f = jax.nn.sigmoid(
            gi_f[:, :H] + lax.dot_general(h_f, hfv_ref[:H], dn)
            + bhh_f_ref[:, :H])
        r_b = jax.nn.sigmoid(
            gi_b[:, :H] + lax.dot_general(h_b, hbv_ref[:H], dn)
            + bhh_b_ref[:, :H])
        z_f = jax.nn.sigmoid(
            gi_f[:, H:2 * H] + lax.dot_general(h_f, hfv_ref[H:2 * H], dn)
            + bhh_f_ref[:, H:2 * H])
        z_b = jax.nn.sigmoid(
            gi_b[:, H:2 * H] + lax.dot_general(h_b, hbv_ref[H:2 * H], dn)
            + bhh_b_ref[:, H:2 * H])
        n_f = jnp.tanh(
            gi_f[:, 2 * H:]
            + r_f * (lax.dot_general(h_f, hfv_ref[2 * H:], dn)
                     + bhh_f_ref[:, 2 * H:]))
        n_b = jnp.tanh(
            gi_b[:, 2 * H:]
            + r_b * (lax.dot_general(h_b, hbv_ref[2 * H:], dn)
                     + bhh_b_ref[:, 2 * H:]))
        h_f = (1.0 - z_f) * n_f + z_f * h_f
        h_b = (1.0 - z_b) * n_b + z_b * h_b
        out_ref[:, pl.ds(t, 1)] = h_f[:, None]
        bwd_ref[:, pl.ds(S - 1 - t, 1)] = h_b[:, None]
        return h_f, h_b

    def step4(u, carry):
        carry = step(4 * u + 1, step(4 * u, carry))
        return step(4 * u + 3, step(4 * u + 2, carry))

    h0 = jnp.zeros((B, MEM_DIM), jnp.float32)
    lax.fori_loop(0, S // 4, step4, (h0, h0))
    out_ref[...] += bwd_ref[...]


def _tc_call(rows4, mk4,
             ioux_W, iouh_W, ioux_b, iouh_b, wih_f, wih_b, whh_f, whh_b,
             bih_f, bih_b, bhh_f, bhh_b):
    vspec = pl.BlockSpec(memory_space=pltpu.MemorySpace.VMEM)
    hspec = pl.BlockSpec(memory_space=pltpu.MemorySpace.HBM)
    return pl.pallas_call(
        _tc_body,
        out_shape=jax.ShapeDtypeStruct((B, S, MEM_DIM), jnp.float32),
        in_specs=[vspec, vspec, vspec, vspec, vspec, vspec,
                  hspec, hspec, hspec, hspec,
                  vspec, vspec, vspec, vspec],
        scratch_shapes=[
            pltpu.VMEM((S, B, 3 * MEM_DIM), jnp.float32),
            pltpu.VMEM((S, B, 3 * MEM_DIM), jnp.float32),
            pltpu.VMEM((B, S, MEM_DIM), jnp.float32),
            pltpu.VMEM((3 * MEM_DIM, MEM_DIM), jnp.float32),
            pltpu.VMEM((3 * MEM_DIM, MEM_DIM), jnp.float32),
            pltpu.VMEM((3 * MEM_DIM, MEM_DIM), jnp.float32),
            pltpu.VMEM((3 * MEM_DIM, MEM_DIM), jnp.float32),
            pltpu.SemaphoreType.DMA,
            pltpu.SemaphoreType.DMA,
        ],
    )(rows4, mk4,
      ioux_W, iouh_W, ioux_b, iouh_b, wih_f, wih_b, whh_f, whh_b,
      bih_f, bih_b, bhh_f, bhh_b)


def kernel(embed, leaf_word_idx, child_word_idx, child_idx, contexts_idx,
           ioux_W, ioux_b, iouh_W, iouh_b, fx_W, fx_b, fh_W, fh_b,
           Wih_f, Whh_f, bih_f, bhh_f, Wih_b, Whh_b, bih_b, bhh_b):
    rows_blk, mk_flat = _sc_gather_make()(
        contexts_idx.reshape(-1).astype(jnp.int32),
        leaf_word_idx.astype(jnp.int32),
        child_word_idx.astype(jnp.int32),
        child_idx.astype(jnp.int32).reshape(-1), embed)

    out = _tc_call(
        rows_blk.reshape(NWORKERS, 3, PER_W, IN_DIM), mk_flat.reshape(P, 4),
        ioux_W, iouh_W,
        ioux_b.reshape(1, 3 * MEM_DIM), iouh_b.reshape(1, 3 * MEM_DIM),
        Wih_f, Wih_b, Whh_f, Whh_b,
        bih_f.reshape(1, 3 * MEM_DIM), bih_b.reshape(1, 3 * MEM_DIM),
        bhh_f.reshape(1, 3 * MEM_DIM), bhh_b.reshape(1, 3 * MEM_DIM))
    return out


# speedup vs baseline: 1.3123x; 1.0052x over previous
import functools

import jax
import jax.numpy as jnp
from jax import lax
from jax.experimental import pallas as pl
from jax.experimental.pallas import tpu as pltpu
from jax.experimental.pallas import tpu_sc as plsc

MEM_DIM = 512
IN_DIM = 512
N_LEAF = 8192
N_CHILD = 4096
B = 16
S = 32
P = B * S
NWORKERS = 32
PER_W = P // NWORKERS

@functools.cache
def _sc_gather_make():
    mesh = plsc.VectorSubcoreMesh(core_axis_name="c", subcore_axis_name="s")
    f32 = jnp.float32
    i32 = jnp.int32
    out_type = (
        jax.ShapeDtypeStruct((3 * P, IN_DIM), f32),
        jax.ShapeDtypeStruct((4 * P,), f32),
    )
    scratch = [
        pltpu.VMEM((PER_W,), i32),
        pltpu.VMEM((PER_W,), i32),
        pltpu.VMEM((PER_W,), i32),
        pltpu.VMEM((PER_W,), i32),
        pltpu.VMEM((PER_W,), i32),
        pltpu.VMEM((PER_W,), i32),
        pltpu.VMEM((PER_W,), i32),
        pltpu.VMEM((3 * PER_W,), i32),
        pltpu.VMEM((3 * PER_W, IN_DIM), f32),
        pltpu.VMEM((4 * PER_W,), f32),
        pltpu.SemaphoreType.DMA,
        pltpu.SemaphoreType.DMA,
    ]

    @functools.partial(pl.kernel, mesh=mesh, out_type=out_type,
                       scratch_types=scratch,
                       compiler_params=pltpu.CompilerParams(
                           needs_layout_passes=False))
    def sc_gather(ctx_hbm, lw_hbm, cw_hbm, ci_hbm, embed_hbm,
                  rows_out, mk_out,
                  cidx_v, cwv_v, lwv_v, ci0_v, ci1_v, w0_v, w1_v,
                  ri_v, rows_v, mk_v, sem, osem):
        wid = lax.axis_index("s") * 2 + lax.axis_index("c")
        base = wid * PER_W
        iota = lax.iota(jnp.int32, PER_W)
        pltpu.async_copy(ctx_hbm.at[iota * S + wid], cidx_v, sem).wait()

        cidx = cidx_v[...]
        is_child = cidx > N_LEAF
        is_leaf = (cidx > 0) & (cidx <= N_LEAF)
        n_safe = jnp.where(is_child, cidx - (1 + N_LEAF), 0)
        leaf_i = jnp.where(is_leaf, cidx - 1, 0)
        g1 = pltpu.async_copy(cw_hbm.at[n_safe], cwv_v, sem)
        g2 = pltpu.async_copy(lw_hbm.at[leaf_i], lwv_v, sem)
        g3 = pltpu.async_copy(ci_hbm.at[n_safe], ci0_v, sem)
        g4 = pltpu.async_copy(ci_hbm.at[N_CHILD + n_safe], ci1_v, sem)
        g1.wait()
        g2.wait()
        g3.wait()
        g4.wait()

        wx = jnp.where(is_child, cwv_v[...], lwv_v[...])
        ci0 = ci0_v[...]
        ci1 = ci1_v[...]
        m0 = is_child & (ci0 > 0)
        m1 = is_child & (ci1 > 0)
        g5 = pltpu.async_copy(lw_hbm.at[jnp.where(m0, ci0 - 1, 0)], w0_v, sem)
        g6 = pltpu.async_copy(lw_hbm.at[jnp.where(m1, ci1 - 1, 0)], w1_v, sem)
        ri_v[pl.ds(0, PER_W)] = wx

        one = jnp.float32(1.0)
        zero = jnp.float32(0.0)
        slot = iota * 4
        plsc.store_scatter(mk_v, [slot], jnp.where(is_child | is_leaf, one, zero))
        plsc.store_scatter(mk_v, [slot + 1], jnp.where(m0, one, zero))
        plsc.store_scatter(mk_v, [slot + 2], jnp.where(m1, one, zero))
        plsc.store_scatter(mk_v, [slot + 3], jnp.where(is_child, one, zero))
        cm = pltpu.async_copy(mk_v, mk_out.at[pl.ds(4 * base, 4 * PER_W)], osem)

        g5.wait()
        g6.wait()
        ri_v[pl.ds(PER_W, PER_W)] = w0_v[...]
        ri_v[pl.ds(2 * PER_W, PER_W)] = w1_v[...]
        pltpu.async_copy(embed_hbm.at[ri_v], rows_v, sem).wait()
        co = pltpu.async_copy(rows_v, rows_out.at[pl.ds(3 * base, 3 * PER_W)],
                              osem)
        cm.wait()
        co.wait()

    return sc_gather


def _tc_body(rows_ref, mk_ref,
             ioux_W_ref, iouh_W_ref, ioux_b_ref, iouh_b_ref,
             wih_f_ref, wih_b_ref, whh_f_ref, whh_b_ref,
             bih_f_ref, bih_b_ref, bhh_f_ref, bhh_b_ref,
             out_ref, gif_ref, gib_ref, bwd_ref,
             wfv_ref, wbv_ref, hfv_ref, hbv_ref, wsem, hsem):
    H = MEM_DIM
    dn = (((1,), (1,)), ((), ()))

    cw1 = pltpu.async_copy(wih_f_ref, wfv_ref, wsem)
    cw2 = pltpu.async_copy(wih_b_ref, wbv_ref, wsem)
    ch1 = pltpu.async_copy(whh_f_ref, hfv_ref, hsem)
    ch2 = pltpu.async_copy(whh_b_ref, hbv_ref, hsem)

    x_raw = rows_ref[:, 0].reshape(P, IN_DIM)
    c0_raw = rows_ref[:, 1].reshape(P, IN_DIM)
    c1_raw = rows_ref[:, 2].reshape(P, IN_DIM)
    X = x_raw * mk_ref[:, 0:1]
    HS = c0_raw * mk_ref[:, 1:2] + c1_raw * mk_ref[:, 2:3]
    iou = (lax.dot_general(X, ioux_W_ref[...], dn)
           + lax.dot_general(HS, iouh_W_ref[...], dn)
           + (ioux_b_ref[...] + iouh_b_ref[...]))
    i = jax.nn.sigmoid(iou[:, :H])
    o = jax.nn.sigmoid(iou[:, H:2 * H])
    u = jnp.tanh(iou[:, 2 * H:])
    h_op = o * jnp.tanh(i * u)
    mc = mk_ref[:, 3:4]
    enc = mc * h_op + (1.0 - mc) * X

    cw1.wait()
    cw2.wait()
    gif_ref[...] = (lax.dot_general(enc, wfv_ref[...], dn)
                    + bih_f_ref[...]).reshape(S, B, 3 * H)
    gib_ref[...] = (lax.dot_general(enc, wbv_ref[...], dn)
                    + bih_b_ref[...]).reshape(S, B, 3 * H)
    ch1.wait()
    ch2.wait()

    def step(t, carry):
        h_f, h_b = carry
        gi_f = gif_ref[t]
        gi_b = gib_ref[S - 1 - t]
        r_f = jax.nn.sigmoid(
            gi_f[:, :H] + lax.dot_general(h_f, hfv_ref[:H], dn)
            + bhh_f_ref[:, :H])
        r_b = jax.nn.sigmoid(
            gi_b[:, :H] + lax.dot_general(h_b, hbv_ref[:H], dn)
            + bhh_b_ref[:, :H])
        z_f = jax.nn.sigmoid(
            gi_f[:, H:2 * H] + lax.dot_general(h_f, hfv_ref[H:2 * H], dn)
            + bhh_f_ref[:, H:2 * H])
        z_b = jax.nn.sigmoid(
            gi_b[:, H:2 * H] + lax.dot_general(h_b, hbv_ref[H:2 * H], dn)
            + bhh_b_ref[:, H:2 * H])
        n_f = jnp.tanh(
            gi_f[:, 2 * H:]
            + r_f * (lax.dot_general(h_f, hfv_ref[2 * H:], dn)
                     + bhh_f_ref[:, 2 * H:]))
        n_b = jnp.tanh(
            gi_b[:, 2 * H:]
            + r_b * (lax.dot_general(h_b, hbv_ref[2 * H:], dn)
                     + bhh_b_ref[:, 2 * H:]))
        h_f = (1.0 - z_f) * n_f + z_f * h_f
        h_b = (1.0 - z_b) * n_b + z_b * h_b
        out_ref[:, pl.ds(t, 1)] = h_f[:, None]
        bwd_ref[:, pl.ds(S - 1 - t, 1)] = h_b[:, None]
        return h_f, h_b

    def step8(u, carry):
        for j in range(8):
            carry = step(8 * u + j, carry)
        return carry

    h0 = jnp.zeros((B, MEM_DIM), jnp.float32)
    lax.fori_loop(0, S // 8, step8, (h0, h0))
    out_ref[...] += bwd_ref[...]


def _tc_call(rows4, mk4,
             ioux_W, iouh_W, ioux_b, iouh_b, wih_f, wih_b, whh_f, whh_b,
             bih_f, bih_b, bhh_f, bhh_b):
    vspec = pl.BlockSpec(memory_space=pltpu.MemorySpace.VMEM)
    hspec = pl.BlockSpec(memory_space=pltpu.MemorySpace.HBM)
    return pl.pallas_call(
        _tc_body,
        out_shape=jax.ShapeDtypeStruct((B, S, MEM_DIM), jnp.float32),
        in_specs=[vspec, vspec, vspec, vspec, vspec, vspec,
                  hspec, hspec, hspec, hspec,
                  vspec, vspec, vspec, vspec],
        scratch_shapes=[
            pltpu.VMEM((S, B, 3 * MEM_DIM), jnp.float32),
            pltpu.VMEM((S, B, 3 * MEM_DIM), jnp.float32),
            pltpu.VMEM((B, S, MEM_DIM), jnp.float32),
            pltpu.VMEM((3 * MEM_DIM, MEM_DIM), jnp.float32),
            pltpu.VMEM((3 * MEM_DIM, MEM_DIM), jnp.float32),
            pltpu.VMEM((3 * MEM_DIM, MEM_DIM), jnp.float32),
            pltpu.VMEM((3 * MEM_DIM, MEM_DIM), jnp.float32),
            pltpu.SemaphoreType.DMA,
            pltpu.SemaphoreType.DMA,
        ],
    )(rows4, mk4,
      ioux_W, iouh_W, ioux_b, iouh_b, wih_f, wih_b, whh_f, whh_b,
      bih_f, bih_b, bhh_f, bhh_b)


def kernel(embed, leaf_word_idx, child_word_idx, child_idx, contexts_idx,
           ioux_W, ioux_b, iouh_W, iouh_b, fx_W, fx_b, fh_W, fh_b,
           Wih_f, Whh_f, bih_f, bhh_f, Wih_b, Whh_b, bih_b, bhh_b):
    rows_blk, mk_flat = _sc_gather_make()(
        contexts_idx.reshape(-1).astype(jnp.int32),
        leaf_word_idx.astype(jnp.int32),
        child_word_idx.astype(jnp.int32),
        child_idx.astype(jnp.int32).reshape(-1), embed)

    out = _tc_call(
        rows_blk.reshape(NWORKERS, 3, PER_W, IN_DIM), mk_flat.reshape(P, 4),
        ioux_W, iouh_W,
        ioux_b.reshape(1, 3 * MEM_DIM), iouh_b.reshape(1, 3 * MEM_DIM),
        Wih_f, Wih_b, Whh_f, Whh_b,
        bih_f.reshape(1, 3 * MEM_DIM), bih_b.reshape(1, 3 * MEM_DIM),
        bhh_f.reshape(1, 3 * MEM_DIM), bhh_b.reshape(1, 3 * MEM_DIM))
    return out
